# Initial kernel scaffold; baseline (speedup 1.0000x reference)
import jax
import jax.numpy as jnp
from jax.experimental import pallas as pl


def kernel(x, Gw_sea, Ew_sea, Eb_sea, Gw_trend, Ew_trend, Eb_trend):
    raise NotImplementedError("write your pallas kernel here")


# speedup vs baseline: 1.4368x; 1.4368x over previous
import functools

import jax
import jax.numpy as jnp
from jax.experimental import pallas as pl
from jax.experimental.pallas import tpu as pltpu

_KERNEL = 25
_PAD = (_KERNEL - 1) // 2
_E = 8
_B, _L, _V = 4, 2048, 256
_P = 1024
_P_BLK = 512


def _top2_coeffs(probs):
    iota = jax.lax.broadcasted_iota(jnp.int32, probs.shape, 0)
    m1 = jnp.max(probs, axis=0, keepdims=True)
    i1 = jnp.min(jnp.where(probs == m1, iota, _E), axis=0, keepdims=True)
    mask1 = iota == i1
    p2 = jnp.where(mask1, -jnp.inf, probs)
    m2 = jnp.max(p2, axis=0, keepdims=True)
    i2 = jnp.min(jnp.where(p2 == m2, iota, _E), axis=0, keepdims=True)
    mask2 = iota == i2
    return probs * (mask1.astype(probs.dtype) + mask2.astype(probs.dtype))


def _softmax0(logits):
    z = logits - jnp.max(logits, axis=0, keepdims=True)
    ez = jnp.exp(z)
    return ez / jnp.sum(ez, axis=0, keepdims=True)


def _decomp_route_kernel(x_ref, gws_ref, gwt_ref,
                         sea_ref, trend_ref, cs_ref, ct_ref, ptm_ref):
    b = pl.program_id(0)
    x = x_ref[0]
    front = jnp.broadcast_to(x[0:1, :], (_PAD, _V))
    back = jnp.broadcast_to(x[_L - 1:_L, :], (_PAD, _V))
    xp = jnp.concatenate([front, x, back], axis=0)
    acc = xp[0:_L, :]
    for k in range(1, _KERNEL):
        acc = acc + xp[k:k + _L, :]
    mov = acc * (1.0 / _KERNEL)
    sea = x - mov

    logits_s = jnp.dot(gws_ref[...], sea, preferred_element_type=jnp.float32)
    logits_t = jnp.dot(gwt_ref[...], mov, preferred_element_type=jnp.float32)
    probs_s = _softmax0(logits_s)
    probs_t = _softmax0(logits_t)

    cs_ref[0] = _top2_coeffs(probs_s)
    ct_ref[0] = _top2_coeffs(probs_t)
    sea_ref[0] = sea.astype(jnp.bfloat16)
    trend_ref[0] = mov.astype(jnp.bfloat16)

    @pl.when(b == 0)
    def _():
        ptm_ref[...] = jnp.zeros_like(ptm_ref)

    ptm_ref[...] += probs_t * (1.0 / _B)


def _expert_kernel(cs_ref, ct_ref, ebs_ref, ebt_ref,
                   ews_ref, ewt_ref, sea_ref, trend_ref,
                   out_ref):
    e = pl.program_id(0)
    pb = pl.program_id(1)
    b = pl.program_id(2)

    ws = ews_ref[0].astype(jnp.bfloat16)
    wt = ewt_ref[0].astype(jnp.bfloat16)
    ys = jnp.dot(ws, sea_ref[0], preferred_element_type=jnp.float32)
    yt = jnp.dot(wt, trend_ref[0], preferred_element_type=jnp.float32)

    cs_row = cs_ref[b, e, :][None, :]
    ct_row = ct_ref[b, e, :][None, :]
    ebs_col = ebs_ref[e, pl.ds(pb * _P_BLK, _P_BLK)][:, None]
    ebt_col = ebt_ref[e, pl.ds(pb * _P_BLK, _P_BLK)][:, None]

    contrib = cs_row * (ys + ebs_col) + ct_row * (yt + ebt_col)

    slot = (b, pl.ds(pb * _P_BLK, _P_BLK), slice(None))

    @pl.when(e == 0)
    def _():
        out_ref[slot] = contrib

    @pl.when(e > 0)
    def _():
        out_ref[slot] += contrib


@jax.jit
def kernel(x, Gw_sea, Ew_sea, Eb_sea, Gw_trend, Ew_trend, Eb_trend):
    sea_bf, trend_bf, c_sea, c_trend, ptm = pl.pallas_call(
        _decomp_route_kernel,
        grid=(_B,),
        in_specs=[
            pl.BlockSpec((1, _L, _V), lambda b: (b, 0, 0)),
            pl.BlockSpec((_E, _L), lambda b: (0, 0)),
            pl.BlockSpec((_E, _L), lambda b: (0, 0)),
        ],
        out_specs=[
            pl.BlockSpec((1, _L, _V), lambda b: (b, 0, 0)),
            pl.BlockSpec((1, _L, _V), lambda b: (b, 0, 0)),
            pl.BlockSpec((1, _E, _V), lambda b: (b, 0, 0)),
            pl.BlockSpec((1, _E, _V), lambda b: (b, 0, 0)),
            pl.BlockSpec((_E, _V), lambda b: (0, 0)),
        ],
        out_shape=[
            jax.ShapeDtypeStruct((_B, _L, _V), jnp.bfloat16),
            jax.ShapeDtypeStruct((_B, _L, _V), jnp.bfloat16),
            jax.ShapeDtypeStruct((_B, _E, _V), jnp.float32),
            jax.ShapeDtypeStruct((_B, _E, _V), jnp.float32),
            jax.ShapeDtypeStruct((_E, _V), jnp.float32),
        ],
        compiler_params=pltpu.CompilerParams(
            dimension_semantics=("arbitrary",),
        ),
    )(x, Gw_sea, Gw_trend)

    out = pl.pallas_call(
        _expert_kernel,
        grid=(_E, _P // _P_BLK, _B),
        in_specs=[
            pl.BlockSpec((_B, _E, _V), lambda e, pb, b: (0, 0, 0)),
            pl.BlockSpec((_B, _E, _V), lambda e, pb, b: (0, 0, 0)),
            pl.BlockSpec((_E, _P), lambda e, pb, b: (0, 0)),
            pl.BlockSpec((_E, _P), lambda e, pb, b: (0, 0)),
            pl.BlockSpec((1, _P_BLK, _L), lambda e, pb, b: (e, pb, 0)),
            pl.BlockSpec((1, _P_BLK, _L), lambda e, pb, b: (e, pb, 0)),
            pl.BlockSpec((1, _L, _V), lambda e, pb, b: (b, 0, 0)),
            pl.BlockSpec((1, _L, _V), lambda e, pb, b: (b, 0, 0)),
        ],
        out_specs=pl.BlockSpec((_B, _P, _V), lambda e, pb, b: (0, 0, 0)),
        out_shape=jax.ShapeDtypeStruct((_B, _P, _V), jnp.float32),
        compiler_params=pltpu.CompilerParams(
            dimension_semantics=("arbitrary", "arbitrary", "arbitrary"),
        ),
    )(c_sea, c_trend, Eb_sea, Eb_trend, Ew_sea, Ew_trend, sea_bf, trend_bf)

    return out, jnp.transpose(ptm, (1, 0))


# speedup vs baseline: 1.5429x; 1.0738x over previous
import functools

import jax
import jax.numpy as jnp
from jax.experimental import pallas as pl
from jax.experimental.pallas import tpu as pltpu

_KERNEL = 25
_PAD = (_KERNEL - 1) // 2
_E = 8
_B, _L, _V = 4, 2048, 256
_P = 1024
_P_BLK = 512


def _top2_coeffs(probs):
    iota = jax.lax.broadcasted_iota(jnp.int32, probs.shape, 0)
    m1 = jnp.max(probs, axis=0, keepdims=True)
    i1 = jnp.min(jnp.where(probs == m1, iota, _E), axis=0, keepdims=True)
    mask1 = iota == i1
    p2 = jnp.where(mask1, -jnp.inf, probs)
    m2 = jnp.max(p2, axis=0, keepdims=True)
    i2 = jnp.min(jnp.where(p2 == m2, iota, _E), axis=0, keepdims=True)
    mask2 = iota == i2
    return probs * (mask1.astype(probs.dtype) + mask2.astype(probs.dtype))


def _softmax0(logits):
    z = logits - jnp.max(logits, axis=0, keepdims=True)
    ez = jnp.exp(z)
    return ez / jnp.sum(ez, axis=0, keepdims=True)


def _decomp_route_kernel(x_ref, gws_ref, gwt_ref,
                         sea_ref, trend_ref, cs_ref, ct_ref, ptm_ref):
    b = pl.program_id(0)
    x = x_ref[0]
    front = jnp.broadcast_to(x[0:1, :], (_PAD, _V))
    back = jnp.broadcast_to(x[_L - 1:_L, :], (_PAD, _V))
    xp = jnp.concatenate([front, x, back], axis=0)
    acc = xp[0:_L, :]
    for k in range(1, _KERNEL):
        acc = acc + xp[k:k + _L, :]
    mov = acc * (1.0 / _KERNEL)
    sea = x - mov

    logits_s = jnp.dot(gws_ref[...], sea, preferred_element_type=jnp.float32)
    logits_t = jnp.dot(gwt_ref[...], mov, preferred_element_type=jnp.float32)
    probs_s = _softmax0(logits_s)
    probs_t = _softmax0(logits_t)

    cs_ref[0] = _top2_coeffs(probs_s)
    ct_ref[0] = _top2_coeffs(probs_t)
    sea_ref[0] = sea.astype(jnp.bfloat16)
    trend_ref[0] = mov.astype(jnp.bfloat16)

    @pl.when(b == 0)
    def _():
        ptm_ref[...] = jnp.zeros_like(ptm_ref)

    ptm_ref[...] += probs_t * (1.0 / _B)


def _expert_kernel(cs_ref, ct_ref, ebs_ref, ebt_ref,
                   ews_ref, ewt_ref, sea_ref, trend_ref,
                   out_ref):
    e = pl.program_id(0)
    pb = pl.program_id(1)
    b = pl.program_id(2)

    ws = ews_ref[0].astype(jnp.bfloat16)
    wt = ewt_ref[0].astype(jnp.bfloat16)
    ys = jnp.dot(ws, sea_ref[b], preferred_element_type=jnp.float32)
    yt = jnp.dot(wt, trend_ref[b], preferred_element_type=jnp.float32)

    cs_row = cs_ref[b, e, :][None, :]
    ct_row = ct_ref[b, e, :][None, :]
    ebs_col = ebs_ref[e, pl.ds(pb * _P_BLK, _P_BLK)][:, None]
    ebt_col = ebt_ref[e, pl.ds(pb * _P_BLK, _P_BLK)][:, None]

    contrib = cs_row * (ys + ebs_col) + ct_row * (yt + ebt_col)

    slot = (b, pl.ds(pb * _P_BLK, _P_BLK), slice(None))

    @pl.when(e == 0)
    def _():
        out_ref[slot] = contrib

    @pl.when(e > 0)
    def _():
        out_ref[slot] += contrib


@jax.jit
def kernel(x, Gw_sea, Ew_sea, Eb_sea, Gw_trend, Ew_trend, Eb_trend):
    sea_bf, trend_bf, c_sea, c_trend, ptm = pl.pallas_call(
        _decomp_route_kernel,
        grid=(_B,),
        in_specs=[
            pl.BlockSpec((1, _L, _V), lambda b: (b, 0, 0)),
            pl.BlockSpec((_E, _L), lambda b: (0, 0)),
            pl.BlockSpec((_E, _L), lambda b: (0, 0)),
        ],
        out_specs=[
            pl.BlockSpec((1, _L, _V), lambda b: (b, 0, 0)),
            pl.BlockSpec((1, _L, _V), lambda b: (b, 0, 0)),
            pl.BlockSpec((1, _E, _V), lambda b: (b, 0, 0)),
            pl.BlockSpec((1, _E, _V), lambda b: (b, 0, 0)),
            pl.BlockSpec((_E, _V), lambda b: (0, 0)),
        ],
        out_shape=[
            jax.ShapeDtypeStruct((_B, _L, _V), jnp.bfloat16),
            jax.ShapeDtypeStruct((_B, _L, _V), jnp.bfloat16),
            jax.ShapeDtypeStruct((_B, _E, _V), jnp.float32),
            jax.ShapeDtypeStruct((_B, _E, _V), jnp.float32),
            jax.ShapeDtypeStruct((_E, _V), jnp.float32),
        ],
        compiler_params=pltpu.CompilerParams(
            dimension_semantics=("arbitrary",),
        ),
    )(x, Gw_sea, Gw_trend)

    out = pl.pallas_call(
        _expert_kernel,
        grid=(_E, _P // _P_BLK, _B),
        in_specs=[
            pl.BlockSpec((_B, _E, _V), lambda e, pb, b: (0, 0, 0)),
            pl.BlockSpec((_B, _E, _V), lambda e, pb, b: (0, 0, 0)),
            pl.BlockSpec((_E, _P), lambda e, pb, b: (0, 0)),
            pl.BlockSpec((_E, _P), lambda e, pb, b: (0, 0)),
            pl.BlockSpec((1, _P_BLK, _L), lambda e, pb, b: (e, pb, 0)),
            pl.BlockSpec((1, _P_BLK, _L), lambda e, pb, b: (e, pb, 0)),
            pl.BlockSpec((_B, _L, _V), lambda e, pb, b: (0, 0, 0)),
            pl.BlockSpec((_B, _L, _V), lambda e, pb, b: (0, 0, 0)),
        ],
        out_specs=pl.BlockSpec((_B, _P, _V), lambda e, pb, b: (0, 0, 0)),
        out_shape=jax.ShapeDtypeStruct((_B, _P, _V), jnp.float32),
        compiler_params=pltpu.CompilerParams(
            dimension_semantics=("arbitrary", "arbitrary", "arbitrary"),
        ),
    )(c_sea, c_trend, Eb_sea, Eb_trend, Ew_sea, Ew_trend, sea_bf, trend_bf)

    return out, jnp.transpose(ptm, (1, 0))


# speedup vs baseline: 1.8907x; 1.2255x over previous
import jax
import jax.numpy as jnp
from jax import lax
from jax.experimental import pallas as pl
from jax.experimental.pallas import tpu as pltpu

_KERNEL = 25
_PAD = (_KERNEL - 1) // 2
_E = 8
_B, _L, _V = 4, 2048, 256
_BV = _B * _V
_P = 1024
_P_BLK = 512


def _top2_coeffs(probs):
    iota = jax.lax.broadcasted_iota(jnp.int32, probs.shape, 0)
    m1 = jnp.max(probs, axis=0, keepdims=True)
    i1 = jnp.min(jnp.where(probs == m1, iota, _E), axis=0, keepdims=True)
    mask1 = iota == i1
    p2 = jnp.where(mask1, -jnp.inf, probs)
    m2 = jnp.max(p2, axis=0, keepdims=True)
    i2 = jnp.min(jnp.where(p2 == m2, iota, _E), axis=0, keepdims=True)
    mask2 = iota == i2
    return probs * (mask1.astype(probs.dtype) + mask2.astype(probs.dtype))


def _softmax0(logits):
    z = logits - jnp.max(logits, axis=0, keepdims=True)
    ez = jnp.exp(z)
    return ez / jnp.sum(ez, axis=0, keepdims=True)


def _decomp_route_kernel(x_ref, gws_ref, gwt_ref,
                         sea_ref, trend_ref, cs_ref, ct_ref, ptm_ref):
    b = pl.program_id(0)
    x = x_ref[0]
    front = jnp.broadcast_to(x[0:1, :], (_PAD, _V))
    back = jnp.broadcast_to(x[_L - 1:_L, :], (_PAD, _V))
    xp = jnp.concatenate([front, x, back], axis=0)
    acc = xp[0:_L, :]
    for k in range(1, _KERNEL):
        acc = acc + xp[k:k + _L, :]
    mov = acc * (1.0 / _KERNEL)
    sea = x - mov

    logits_s = jnp.dot(gws_ref[...], sea, preferred_element_type=jnp.float32,
                       precision=lax.Precision.HIGHEST)
    logits_t = jnp.dot(gwt_ref[...], mov, preferred_element_type=jnp.float32,
                       precision=lax.Precision.HIGHEST)
    probs_s = _softmax0(logits_s)
    probs_t = _softmax0(logits_t)

    cs_ref[...] = _top2_coeffs(probs_s)
    ct_ref[...] = _top2_coeffs(probs_t)
    sea_ref[...] = sea
    trend_ref[...] = mov

    @pl.when(b == 0)
    def _():
        ptm_ref[...] = jnp.zeros_like(ptm_ref)

    ptm_ref[...] += probs_t * (1.0 / _B)


def _expert_kernel(cs_ref, ct_ref, ebs_ref, ebt_ref,
                   ews_ref, ewt_ref, sea_ref, trend_ref,
                   out_ref):
    e = pl.program_id(0)
    pb = pl.program_id(1)

    ys = jnp.dot(ews_ref[0], sea_ref[...], preferred_element_type=jnp.float32,
                 precision=lax.Precision.DEFAULT)
    yt = jnp.dot(ewt_ref[0], trend_ref[...], preferred_element_type=jnp.float32,
                 precision=lax.Precision.DEFAULT)

    cs_row = cs_ref[e, :][None, :]
    ct_row = ct_ref[e, :][None, :]
    contrib = cs_row * ys + ct_row * yt

    dn = (((0,), (0,)), ((), ()))

    @pl.when(e == 0)
    def _():
        ebs_blk = ebs_ref[:, pl.ds(pb * _P_BLK, _P_BLK)]
        ebt_blk = ebt_ref[:, pl.ds(pb * _P_BLK, _P_BLK)]
        bias = (lax.dot_general(ebs_blk, cs_ref[...], dn,
                                preferred_element_type=jnp.float32,
                                precision=lax.Precision.HIGHEST)
                + lax.dot_general(ebt_blk, ct_ref[...], dn,
                                  preferred_element_type=jnp.float32,
                                  precision=lax.Precision.HIGHEST))
        total = contrib + bias
        for b in range(_B):
            out_ref[b, pl.ds(pb * _P_BLK, _P_BLK), :] = total[:, b * _V:(b + 1) * _V]

    @pl.when(e > 0)
    def _():
        for b in range(_B):
            out_ref[b, pl.ds(pb * _P_BLK, _P_BLK), :] += contrib[:, b * _V:(b + 1) * _V]


@jax.jit
def kernel(x, Gw_sea, Ew_sea, Eb_sea, Gw_trend, Ew_trend, Eb_trend):
    sea_cat, trend_cat, c_sea, c_trend, ptm = pl.pallas_call(
        _decomp_route_kernel,
        grid=(_B,),
        in_specs=[
            pl.BlockSpec((1, _L, _V), lambda b: (b, 0, 0)),
            pl.BlockSpec((_E, _L), lambda b: (0, 0)),
            pl.BlockSpec((_E, _L), lambda b: (0, 0)),
        ],
        out_specs=[
            pl.BlockSpec((_L, _V), lambda b: (0, b)),
            pl.BlockSpec((_L, _V), lambda b: (0, b)),
            pl.BlockSpec((_E, _V), lambda b: (0, b)),
            pl.BlockSpec((_E, _V), lambda b: (0, b)),
            pl.BlockSpec((_E, _V), lambda b: (0, 0)),
        ],
        out_shape=[
            jax.ShapeDtypeStruct((_L, _BV), jnp.float32),
            jax.ShapeDtypeStruct((_L, _BV), jnp.float32),
            jax.ShapeDtypeStruct((_E, _BV), jnp.float32),
            jax.ShapeDtypeStruct((_E, _BV), jnp.float32),
            jax.ShapeDtypeStruct((_E, _V), jnp.float32),
        ],
        compiler_params=pltpu.CompilerParams(
            dimension_semantics=("arbitrary",),
        ),
    )(x, Gw_sea, Gw_trend)

    out = pl.pallas_call(
        _expert_kernel,
        grid=(_E, _P // _P_BLK),
        in_specs=[
            pl.BlockSpec((_E, _BV), lambda e, pb: (0, 0)),
            pl.BlockSpec((_E, _BV), lambda e, pb: (0, 0)),
            pl.BlockSpec((_E, _P), lambda e, pb: (0, 0)),
            pl.BlockSpec((_E, _P), lambda e, pb: (0, 0)),
            pl.BlockSpec((1, _P_BLK, _L), lambda e, pb: (e, pb, 0)),
            pl.BlockSpec((1, _P_BLK, _L), lambda e, pb: (e, pb, 0)),
            pl.BlockSpec((_L, _BV), lambda e, pb: (0, 0)),
            pl.BlockSpec((_L, _BV), lambda e, pb: (0, 0)),
        ],
        out_specs=pl.BlockSpec((_B, _P, _V), lambda e, pb: (0, 0, 0)),
        out_shape=jax.ShapeDtypeStruct((_B, _P, _V), jnp.float32),
        compiler_params=pltpu.CompilerParams(
            dimension_semantics=("arbitrary", "arbitrary"),
        ),
    )(c_sea, c_trend, Eb_sea, Eb_trend, Ew_sea, Ew_trend, sea_cat, trend_cat)

    return out, jnp.transpose(ptm, (1, 0))


# speedup vs baseline: 1.9399x; 1.0260x over previous
import jax
import jax.numpy as jnp
from jax import lax
from jax.experimental import pallas as pl
from jax.experimental.pallas import tpu as pltpu

_KERNEL = 25
_PAD = (_KERNEL - 1) // 2
_E = 8
_B, _L, _V = 4, 2048, 256
_BV = _B * _V
_P = 1024
_P_BLK = 512


def _top2_coeffs(probs):
    iota = jax.lax.broadcasted_iota(jnp.int32, probs.shape, 0)
    m1 = jnp.max(probs, axis=0, keepdims=True)
    i1 = jnp.min(jnp.where(probs == m1, iota, _E), axis=0, keepdims=True)
    mask1 = iota == i1
    p2 = jnp.where(mask1, -jnp.inf, probs)
    m2 = jnp.max(p2, axis=0, keepdims=True)
    i2 = jnp.min(jnp.where(p2 == m2, iota, _E), axis=0, keepdims=True)
    mask2 = iota == i2
    return probs * (mask1.astype(probs.dtype) + mask2.astype(probs.dtype))


def _softmax0(logits):
    z = logits - jnp.max(logits, axis=0, keepdims=True)
    ez = jnp.exp(z)
    return ez / jnp.sum(ez, axis=0, keepdims=True)


def _decomp_route_kernel(x_ref, gws_ref, gwt_ref,
                         sea_ref, trend_ref, cs_ref, ct_ref, ptm_ref):
    b = pl.program_id(0)
    x = x_ref[0]
    front = jnp.broadcast_to(x[0:1, :], (_PAD, _V))
    back = jnp.broadcast_to(x[_L - 1:_L, :], (_PAD, _V))
    xp = jnp.concatenate([front, x, back], axis=0)
    acc = xp[0:_L, :]
    for k in range(1, _KERNEL):
        acc = acc + xp[k:k + _L, :]
    mov = acc * (1.0 / _KERNEL)
    sea = x - mov

    logits_s = jnp.dot(gws_ref[...], sea, preferred_element_type=jnp.float32,
                       precision=lax.Precision.HIGHEST)
    logits_t = jnp.dot(gwt_ref[...], mov, preferred_element_type=jnp.float32,
                       precision=lax.Precision.HIGHEST)
    probs_s = _softmax0(logits_s)
    probs_t = _softmax0(logits_t)

    cs_ref[...] = _top2_coeffs(probs_s)
    ct_ref[...] = _top2_coeffs(probs_t)
    sea_ref[...] = sea.astype(jnp.bfloat16)
    trend_ref[...] = mov.astype(jnp.bfloat16)

    @pl.when(b == 0)
    def _():
        ptm_ref[...] = jnp.zeros_like(ptm_ref)

    ptm_ref[...] += probs_t * (1.0 / _B)


def _expert_kernel(cs_ref, ct_ref, ebs_ref, ebt_ref,
                   ews_ref, ewt_ref, sea_ref, trend_ref,
                   out_ref):
    e = pl.program_id(0)
    pb = pl.program_id(1)

    ws = ews_ref[0].astype(jnp.bfloat16)
    wt = ewt_ref[0].astype(jnp.bfloat16)
    ys = jnp.dot(ws, sea_ref[...], preferred_element_type=jnp.float32)
    yt = jnp.dot(wt, trend_ref[...], preferred_element_type=jnp.float32)

    cs_row = cs_ref[e, :][None, :]
    ct_row = ct_ref[e, :][None, :]
    contrib = cs_row * ys + ct_row * yt

    dn = (((0,), (0,)), ((), ()))

    @pl.when(e == 0)
    def _():
        ebs_blk = ebs_ref[:, pl.ds(pb * _P_BLK, _P_BLK)]
        ebt_blk = ebt_ref[:, pl.ds(pb * _P_BLK, _P_BLK)]
        bias = (lax.dot_general(ebs_blk, cs_ref[...], dn,
                                preferred_element_type=jnp.float32,
                                precision=lax.Precision.HIGHEST)
                + lax.dot_general(ebt_blk, ct_ref[...], dn,
                                  preferred_element_type=jnp.float32,
                                  precision=lax.Precision.HIGHEST))
        total = contrib + bias
        for b in range(_B):
            out_ref[b, pl.ds(pb * _P_BLK, _P_BLK), :] = total[:, b * _V:(b + 1) * _V]

    @pl.when(e > 0)
    def _():
        for b in range(_B):
            out_ref[b, pl.ds(pb * _P_BLK, _P_BLK), :] += contrib[:, b * _V:(b + 1) * _V]


@jax.jit
def kernel(x, Gw_sea, Ew_sea, Eb_sea, Gw_trend, Ew_trend, Eb_trend):
    sea_cat, trend_cat, c_sea, c_trend, ptm = pl.pallas_call(
        _decomp_route_kernel,
        grid=(_B,),
        in_specs=[
            pl.BlockSpec((1, _L, _V), lambda b: (b, 0, 0)),
            pl.BlockSpec((_E, _L), lambda b: (0, 0)),
            pl.BlockSpec((_E, _L), lambda b: (0, 0)),
        ],
        out_specs=[
            pl.BlockSpec((_L, _V), lambda b: (0, b)),
            pl.BlockSpec((_L, _V), lambda b: (0, b)),
            pl.BlockSpec((_E, _V), lambda b: (0, b)),
            pl.BlockSpec((_E, _V), lambda b: (0, b)),
            pl.BlockSpec((_E, _V), lambda b: (0, 0)),
        ],
        out_shape=[
            jax.ShapeDtypeStruct((_L, _BV), jnp.bfloat16),
            jax.ShapeDtypeStruct((_L, _BV), jnp.bfloat16),
            jax.ShapeDtypeStruct((_E, _BV), jnp.float32),
            jax.ShapeDtypeStruct((_E, _BV), jnp.float32),
            jax.ShapeDtypeStruct((_E, _V), jnp.float32),
        ],
        compiler_params=pltpu.CompilerParams(
            dimension_semantics=("arbitrary",),
        ),
    )(x, Gw_sea, Gw_trend)

    out = pl.pallas_call(
        _expert_kernel,
        grid=(_E, _P // _P_BLK),
        in_specs=[
            pl.BlockSpec((_E, _BV), lambda e, pb: (0, 0)),
            pl.BlockSpec((_E, _BV), lambda e, pb: (0, 0)),
            pl.BlockSpec((_E, _P), lambda e, pb: (0, 0)),
            pl.BlockSpec((_E, _P), lambda e, pb: (0, 0)),
            pl.BlockSpec((1, _P_BLK, _L), lambda e, pb: (e, pb, 0)),
            pl.BlockSpec((1, _P_BLK, _L), lambda e, pb: (e, pb, 0)),
            pl.BlockSpec((_L, _BV), lambda e, pb: (0, 0)),
            pl.BlockSpec((_L, _BV), lambda e, pb: (0, 0)),
        ],
        out_specs=pl.BlockSpec((_B, _P, _V), lambda e, pb: (0, 0, 0)),
        out_shape=jax.ShapeDtypeStruct((_B, _P, _V), jnp.float32),
        compiler_params=pltpu.CompilerParams(
            dimension_semantics=("arbitrary", "arbitrary"),
        ),
    )(c_sea, c_trend, Eb_sea, Eb_trend, Ew_sea, Ew_trend, sea_cat, trend_cat)

    return out, jnp.transpose(ptm, (1, 0))


# speedup vs baseline: 2.1808x; 1.1242x over previous
import jax
import jax.numpy as jnp
from jax import lax
from jax.experimental import pallas as pl
from jax.experimental.pallas import tpu as pltpu

_KERNEL = 25
_PAD = (_KERNEL - 1) // 2
_E = 8
_B, _L, _V = 4, 2048, 256
_BV = _B * _V
_P = 1024
_P_BLK = 512


def _top2_coeffs(probs):
    iota = jax.lax.broadcasted_iota(jnp.int32, probs.shape, 0)
    m1 = jnp.max(probs, axis=0, keepdims=True)
    i1 = jnp.min(jnp.where(probs == m1, iota, _E), axis=0, keepdims=True)
    mask1 = iota == i1
    p2 = jnp.where(mask1, -jnp.inf, probs)
    m2 = jnp.max(p2, axis=0, keepdims=True)
    i2 = jnp.min(jnp.where(p2 == m2, iota, _E), axis=0, keepdims=True)
    mask2 = iota == i2
    return probs * (mask1.astype(probs.dtype) + mask2.astype(probs.dtype))


def _softmax0(logits):
    z = logits - jnp.max(logits, axis=0, keepdims=True)
    ez = jnp.exp(z)
    return ez / jnp.sum(ez, axis=0, keepdims=True)


def _decomp_route_kernel(x_ref, gws_ref, gwt_ref,
                         sea_ref, trend_ref, cs_ref, ct_ref, ptm_ref):
    b = pl.program_id(0)
    x = x_ref[0]
    front = jnp.broadcast_to(x[0:1, :], (_PAD, _V))
    back = jnp.broadcast_to(x[_L - 1:_L, :], (_PAD, _V))
    xp = jnp.concatenate([front, x, back], axis=0)
    acc = xp[0:_L, :]
    for k in range(1, _KERNEL):
        acc = acc + xp[k:k + _L, :]
    mov = acc * (1.0 / _KERNEL)
    sea = x - mov

    logits_s = jnp.dot(gws_ref[...], sea, preferred_element_type=jnp.float32)
    logits_t = jnp.dot(gwt_ref[...], mov, preferred_element_type=jnp.float32)
    probs_s = _softmax0(logits_s)
    probs_t = _softmax0(logits_t)

    cs_ref[...] = _top2_coeffs(probs_s)
    ct_ref[...] = _top2_coeffs(probs_t)
    sea_ref[...] = sea.astype(jnp.bfloat16)
    trend_ref[...] = mov.astype(jnp.bfloat16)

    @pl.when(b == 0)
    def _():
        ptm_ref[...] = jnp.zeros_like(ptm_ref)

    ptm_ref[...] += probs_t * (1.0 / _B)


def _expert_kernel(cs_ref, ct_ref, ebs_ref, ebt_ref,
                   ews_ref, ewt_ref, sea_ref, trend_ref,
                   out_ref):
    e = pl.program_id(0)
    pb = pl.program_id(1)

    ws = ews_ref[0].astype(jnp.bfloat16)
    wt = ewt_ref[0].astype(jnp.bfloat16)
    ys = jnp.dot(ws, sea_ref[...], preferred_element_type=jnp.float32)
    yt = jnp.dot(wt, trend_ref[...], preferred_element_type=jnp.float32)

    cs_row = cs_ref[e, :][None, :]
    ct_row = ct_ref[e, :][None, :]
    ebs_col = ebs_ref[e, pl.ds(pb * _P_BLK, _P_BLK)][:, None]
    ebt_col = ebt_ref[e, pl.ds(pb * _P_BLK, _P_BLK)][:, None]
    contrib = cs_row * (ys + ebs_col) + ct_row * (yt + ebt_col)

    @pl.when(e == 0)
    def _():
        for b in range(_B):
            out_ref[b, pl.ds(pb * _P_BLK, _P_BLK), :] = contrib[:, b * _V:(b + 1) * _V]

    @pl.when(e > 0)
    def _():
        for b in range(_B):
            out_ref[b, pl.ds(pb * _P_BLK, _P_BLK), :] += contrib[:, b * _V:(b + 1) * _V]


@jax.jit
def kernel(x, Gw_sea, Ew_sea, Eb_sea, Gw_trend, Ew_trend, Eb_trend):
    sea_cat, trend_cat, c_sea, c_trend, ptm = pl.pallas_call(
        _decomp_route_kernel,
        grid=(_B,),
        in_specs=[
            pl.BlockSpec((1, _L, _V), lambda b: (b, 0, 0)),
            pl.BlockSpec((_E, _L), lambda b: (0, 0)),
            pl.BlockSpec((_E, _L), lambda b: (0, 0)),
        ],
        out_specs=[
            pl.BlockSpec((_L, _V), lambda b: (0, b)),
            pl.BlockSpec((_L, _V), lambda b: (0, b)),
            pl.BlockSpec((_E, _V), lambda b: (0, b)),
            pl.BlockSpec((_E, _V), lambda b: (0, b)),
            pl.BlockSpec((_E, _V), lambda b: (0, 0)),
        ],
        out_shape=[
            jax.ShapeDtypeStruct((_L, _BV), jnp.bfloat16),
            jax.ShapeDtypeStruct((_L, _BV), jnp.bfloat16),
            jax.ShapeDtypeStruct((_E, _BV), jnp.float32),
            jax.ShapeDtypeStruct((_E, _BV), jnp.float32),
            jax.ShapeDtypeStruct((_E, _V), jnp.float32),
        ],
        compiler_params=pltpu.CompilerParams(
            dimension_semantics=("arbitrary",),
        ),
    )(x, Gw_sea, Gw_trend)

    out = pl.pallas_call(
        _expert_kernel,
        grid=(_E, _P // _P_BLK),
        in_specs=[
            pl.BlockSpec((_E, _BV), lambda e, pb: (0, 0)),
            pl.BlockSpec((_E, _BV), lambda e, pb: (0, 0)),
            pl.BlockSpec((_E, _P), lambda e, pb: (0, 0)),
            pl.BlockSpec((_E, _P), lambda e, pb: (0, 0)),
            pl.BlockSpec((1, _P_BLK, _L), lambda e, pb: (e, pb, 0)),
            pl.BlockSpec((1, _P_BLK, _L), lambda e, pb: (e, pb, 0)),
            pl.BlockSpec((_L, _BV), lambda e, pb: (0, 0)),
            pl.BlockSpec((_L, _BV), lambda e, pb: (0, 0)),
        ],
        out_specs=pl.BlockSpec((_B, _P, _V), lambda e, pb: (0, 0, 0)),
        out_shape=jax.ShapeDtypeStruct((_B, _P, _V), jnp.float32),
        compiler_params=pltpu.CompilerParams(
            dimension_semantics=("arbitrary", "arbitrary"),
        ),
    )(c_sea, c_trend, Eb_sea, Eb_trend, Ew_sea, Ew_trend, sea_cat, trend_cat)

    return out, jnp.transpose(ptm, (1, 0))


# speedup vs baseline: 2.2586x; 1.0356x over previous
import jax
import jax.numpy as jnp
from jax import lax
from jax.experimental import pallas as pl
from jax.experimental.pallas import tpu as pltpu

_KERNEL = 25
_PAD = (_KERNEL - 1) // 2
_E = 8
_B, _L, _V = 4, 2048, 256
_BV = _B * _V
_P = 1024
_P_BLK = 512


def _top2_coeffs(probs):
    iota = jax.lax.broadcasted_iota(jnp.int32, probs.shape, 0)
    m1 = jnp.max(probs, axis=0, keepdims=True)
    i1 = jnp.min(jnp.where(probs == m1, iota, _E), axis=0, keepdims=True)
    mask1 = iota == i1
    p2 = jnp.where(mask1, -jnp.inf, probs)
    m2 = jnp.max(p2, axis=0, keepdims=True)
    i2 = jnp.min(jnp.where(p2 == m2, iota, _E), axis=0, keepdims=True)
    mask2 = iota == i2
    return probs * (mask1.astype(probs.dtype) + mask2.astype(probs.dtype))


def _softmax0(logits):
    z = logits - jnp.max(logits, axis=0, keepdims=True)
    ez = jnp.exp(z)
    return ez / jnp.sum(ez, axis=0, keepdims=True)


def _fused_kernel(x_ref, gws_ref, gwt_ref, ebs_ref, ebt_ref,
                  ews_ref, ewt_ref,
                  out_ref, ptm_ref,
                  sea_sc, trend_sc, cs_sc, ct_sc):
    e = pl.program_id(0)
    pb = pl.program_id(1)

    @pl.when((e == 0) & (pb == 0))
    def _prologue():
        ptm = jnp.zeros((_E, _V), jnp.float32)
        for b in range(_B):
            x = x_ref[b]
            front = jnp.broadcast_to(x[0:1, :], (_PAD, _V))
            back = jnp.broadcast_to(x[_L - 1:_L, :], (_PAD, _V))
            xp = jnp.concatenate([front, x, back], axis=0)
            acc = xp[0:_L, :]
            for k in range(1, _KERNEL):
                acc = acc + xp[k:k + _L, :]
            mov = acc * (1.0 / _KERNEL)
            sea = x - mov

            logits_s = jnp.dot(gws_ref[...], sea,
                               preferred_element_type=jnp.float32)
            logits_t = jnp.dot(gwt_ref[...], mov,
                               preferred_element_type=jnp.float32)
            probs_s = _softmax0(logits_s)
            probs_t = _softmax0(logits_t)

            col = slice(b * _V, (b + 1) * _V)
            cs_sc[:, col] = _top2_coeffs(probs_s)
            ct_sc[:, col] = _top2_coeffs(probs_t)
            sea_sc[:, col] = sea.astype(jnp.bfloat16)
            trend_sc[:, col] = mov.astype(jnp.bfloat16)
            ptm = ptm + probs_t * (1.0 / _B)
        ptm_ref[...] = ptm

    ws = ews_ref[0].astype(jnp.bfloat16)
    wt = ewt_ref[0].astype(jnp.bfloat16)
    ys = jnp.dot(ws, sea_sc[...], preferred_element_type=jnp.float32)
    yt = jnp.dot(wt, trend_sc[...], preferred_element_type=jnp.float32)

    cs_row = cs_sc[e, :][None, :]
    ct_row = ct_sc[e, :][None, :]
    ebs_col = ebs_ref[e, pl.ds(pb * _P_BLK, _P_BLK)][:, None]
    ebt_col = ebt_ref[e, pl.ds(pb * _P_BLK, _P_BLK)][:, None]
    contrib = cs_row * (ys + ebs_col) + ct_row * (yt + ebt_col)

    @pl.when(e == 0)
    def _():
        for b in range(_B):
            out_ref[b, pl.ds(pb * _P_BLK, _P_BLK), :] = contrib[:, b * _V:(b + 1) * _V]

    @pl.when(e > 0)
    def _():
        for b in range(_B):
            out_ref[b, pl.ds(pb * _P_BLK, _P_BLK), :] += contrib[:, b * _V:(b + 1) * _V]


@jax.jit
def kernel(x, Gw_sea, Ew_sea, Eb_sea, Gw_trend, Ew_trend, Eb_trend):
    out, ptm = pl.pallas_call(
        _fused_kernel,
        grid=(_E, _P // _P_BLK),
        in_specs=[
            pl.BlockSpec((_B, _L, _V), lambda e, pb: (0, 0, 0)),
            pl.BlockSpec((_E, _L), lambda e, pb: (0, 0)),
            pl.BlockSpec((_E, _L), lambda e, pb: (0, 0)),
            pl.BlockSpec((_E, _P), lambda e, pb: (0, 0)),
            pl.BlockSpec((_E, _P), lambda e, pb: (0, 0)),
            pl.BlockSpec((1, _P_BLK, _L), lambda e, pb: (e, pb, 0)),
            pl.BlockSpec((1, _P_BLK, _L), lambda e, pb: (e, pb, 0)),
        ],
        out_specs=[
            pl.BlockSpec((_B, _P, _V), lambda e, pb: (0, 0, 0)),
            pl.BlockSpec((_E, _V), lambda e, pb: (0, 0)),
        ],
        out_shape=[
            jax.ShapeDtypeStruct((_B, _P, _V), jnp.float32),
            jax.ShapeDtypeStruct((_E, _V), jnp.float32),
        ],
        scratch_shapes=[
            pltpu.VMEM((_L, _BV), jnp.bfloat16),
            pltpu.VMEM((_L, _BV), jnp.bfloat16),
            pltpu.VMEM((_E, _BV), jnp.float32),
            pltpu.VMEM((_E, _BV), jnp.float32),
        ],
        compiler_params=pltpu.CompilerParams(
            dimension_semantics=("arbitrary", "arbitrary"),
        ),
    )(x, Gw_sea, Gw_trend, Eb_sea, Eb_trend, Ew_sea, Ew_trend)

    return out, jnp.transpose(ptm, (1, 0))
